# Initial kernel scaffold; baseline (speedup 1.0000x reference)
import jax
import jax.numpy as jnp
from jax.experimental import pallas as pl


def kernel(context_ids, target_ids, neg_ids, in_embed, out_embed):
    raise NotImplementedError("write your pallas kernel here")


# speedup vs baseline: 3.2449x; 3.2449x over previous
import functools

import jax
import jax.numpy as jnp
from jax import lax
from jax.experimental import pallas as pl
from jax.experimental.pallas import tpu as pltpu
from jax.experimental.pallas import tpu_sc as plsc

VOCAB = 1000000
D = 64
B = 16384
CTX = 20
NEG = 20

NC = 2
NS = 16
NW = NC * NS
BPW = B // NW
CB = 16
NCHUNK = BPW // CB
IDS_PER_CHUNK = CB * CTX
QROWS = 80
NQ = IDS_PER_CHUNK // QROWS


def _tree_sum(vals):
    vals = list(vals)
    while len(vals) > 1:
        nxt = [vals[i] + vals[i + 1] for i in range(0, len(vals) - 1, 2)]
        if len(vals) % 2:
            nxt.append(vals[-1])
        vals = nxt
    return vals[0]


def _sc_body(ctx_ids2, neg_ids2, tgt_ids, in_embed, out_embed,
             pos_out, neg_out,
             ctx_idx, neg_idx, tgt_idx, ctx_rows, neg_rows, pos_rows,
             pos_stage, neg_stage, sem):
    c = lax.axis_index("c")
    s = lax.axis_index("s")
    wid = s * NC + c
    iota = lax.iota(jnp.int32, 16)

    def chunk_body(i, carry_unused):
        b0 = wid * BPW + i * CB
        r0 = wid * (BPW * CTX // QROWS) + i * NQ

        pltpu.sync_copy(ctx_ids2.at[pl.ds(r0, NQ)], ctx_idx)
        pltpu.sync_copy(neg_ids2.at[pl.ds(r0, NQ)], neg_idx)
        pltpu.sync_copy(tgt_ids.at[pl.ds(b0, CB)], tgt_idx)

        cps = []
        for q in range(NQ):
            cps.append(pltpu.async_copy(
                in_embed.at[ctx_idx.at[q]],
                ctx_rows.at[pl.ds(q * QROWS, QROWS)], sem))
            cps.append(pltpu.async_copy(
                out_embed.at[neg_idx.at[q]],
                neg_rows.at[pl.ds(q * QROWS, QROWS)], sem))
        cps.append(pltpu.async_copy(out_embed.at[tgt_idx], pos_rows, sem))
        for cp in cps:
            cp.wait()

        zero = jnp.zeros((16,), jnp.float32)

        def d_body(d, acc):
            pos_acc = acc[0]
            negacc = acc[1:]
            cold = jnp.full((16,), d, jnp.int32)
            rows_base = iota * CTX
            gath = [plsc.load_gather(ctx_rows, [rows_base + j, cold])
                    for j in range(CTX)]
            vc = _tree_sum(gath) * jnp.float32(1.0 / CTX)
            pu = plsc.load_gather(pos_rows, [iota, cold])
            pos_acc = pos_acc + vc * pu
            negacc = tuple(
                negacc[k] + vc * plsc.load_gather(neg_rows,
                                                  [rows_base + k, cold])
                for k in range(NEG))
            return (pos_acc,) + negacc

        acc = lax.fori_loop(0, D, d_body, (zero,) * (1 + NEG))
        pos_stage[...] = acc[0]
        rows_base = iota * NEG
        for k in range(NEG):
            plsc.store_scatter(neg_stage, [rows_base + k], acc[1 + k])

        pltpu.sync_copy(pos_stage, pos_out.at[pl.ds(b0, CB)])
        pltpu.sync_copy(neg_stage, neg_out.at[pl.ds(b0 * NEG, CB * NEG)])
        return carry_unused

    lax.fori_loop(0, NCHUNK, chunk_body, 0)


@jax.jit
def _sc_call(ctx_ids2, neg_ids2, tgt_ids, in_embed, out_embed):
    mesh = plsc.VectorSubcoreMesh(core_axis_name="c", subcore_axis_name="s")
    f = functools.partial(
        pl.kernel,
        out_type=[
            jax.ShapeDtypeStruct((B,), jnp.float32),
            jax.ShapeDtypeStruct((B * NEG,), jnp.float32),
        ],
        mesh=mesh,
        compiler_params=pltpu.CompilerParams(
            needs_layout_passes=False, use_tc_tiling_on_sc=False),
        scratch_types=[
            pltpu.VMEM((NQ, QROWS), jnp.int32),
            pltpu.VMEM((NQ, QROWS), jnp.int32),
            pltpu.VMEM((CB,), jnp.int32),
            pltpu.VMEM((IDS_PER_CHUNK, D), jnp.float32),
            pltpu.VMEM((IDS_PER_CHUNK, D), jnp.float32),
            pltpu.VMEM((CB, D), jnp.float32),
            pltpu.VMEM((CB,), jnp.float32),
            pltpu.VMEM((CB * NEG,), jnp.float32),
            pltpu.SemaphoreType.DMA,
        ],
    )(_sc_body)
    return f(ctx_ids2, neg_ids2, tgt_ids, in_embed, out_embed)


def kernel(context_ids, target_ids, neg_ids, in_embed, out_embed):
    ctx2 = jnp.asarray(context_ids, jnp.int32).reshape(B * CTX // QROWS, QROWS)
    neg2 = jnp.asarray(neg_ids, jnp.int32).reshape(B * NEG // QROWS, QROWS)
    tgt = jnp.asarray(target_ids, jnp.int32)
    pos, neg_flat = _sc_call(ctx2, neg2, tgt, in_embed, out_embed)
    return pos, neg_flat.reshape(B, NEG)


# speedup vs baseline: 5.2513x; 1.6183x over previous
import functools

import jax
import jax.numpy as jnp
from jax import lax
from jax.experimental import pallas as pl
from jax.experimental.pallas import tpu as pltpu
from jax.experimental.pallas import tpu_sc as plsc

VOCAB = 1000000
D = 64
B = 16384
CTX = 20
NEG = 20
NR = D // 16

NC = 2
NS = 16
NW = NC * NS
BPW = B // NW
CB = 16
NCHUNK = BPW // CB
IDS_PER_CHUNK = CB * CTX
QROWS = 80
NQ = IDS_PER_CHUNK // QROWS
PSTRIDE = 17
KSLOT = CB * PSTRIDE
NK = NEG + 1


def _sc_body(ctx_ids2, neg_ids2, tgt_ids2, in_embed, out_embed,
             pos_out, neg_out,
             ctx_idx, neg_idx, tgt_idx, rows, pbuf, pos_stage, neg_stage,
             rowsem, outsem):
    c = lax.axis_index("c")
    s = lax.axis_index("s")
    wid = s * NC + c
    iota = lax.iota(jnp.int32, 16)
    iota17 = iota * PSTRIDE
    iota20 = iota * NEG

    pltpu.sync_copy(ctx_ids2.at[pl.ds(wid * (BPW * CTX // QROWS),
                                      BPW * CTX // QROWS)], ctx_idx)
    pltpu.sync_copy(neg_ids2.at[pl.ds(wid * (BPW * CTX // QROWS),
                                      BPW * CTX // QROWS)], neg_idx)
    pltpu.sync_copy(tgt_ids2.at[pl.ds(wid * NCHUNK, NCHUNK)], tgt_idx)

    def issue_rows(i):
        p = lax.rem(i, 2)
        r0 = i * NQ
        for q in range(NQ):
            pltpu.async_copy(
                in_embed.at[ctx_idx.at[r0 + q]],
                rows.at[p, pl.ds(q * QROWS, QROWS)], rowsem.at[p])
            pltpu.async_copy(
                out_embed.at[neg_idx.at[r0 + q]],
                rows.at[p, pl.ds(IDS_PER_CHUNK + q * QROWS, QROWS)],
                rowsem.at[p])
        pltpu.async_copy(out_embed.at[tgt_idx.at[i]],
                         rows.at[p, pl.ds(2 * IDS_PER_CHUNK, CB)],
                         rowsem.at[p])

    def drain_rows(i):
        p = lax.rem(i, 2)
        for q in range(NQ):
            pltpu.make_async_copy(
                in_embed.at[ctx_idx.at[0]],
                rows.at[p, pl.ds(q * QROWS, QROWS)], rowsem.at[p]).wait()
            pltpu.make_async_copy(
                out_embed.at[neg_idx.at[0]],
                rows.at[p, pl.ds(IDS_PER_CHUNK + q * QROWS, QROWS)],
                rowsem.at[p]).wait()
        pltpu.make_async_copy(out_embed.at[tgt_idx.at[0]],
                              rows.at[p, pl.ds(2 * IDS_PER_CHUNK, CB)],
                              rowsem.at[p]).wait()

    issue_rows(0)

    def chunk_body(i, carry_unused):
        p = lax.rem(i, 2)
        b0 = wid * BPW + i * CB

        @pl.when(i < NCHUNK - 1)
        def _():
            issue_rows(i + 1)

        drain_rows(i)

        def b_body(b, carry_unused2):
            base = b * CTX
            vc = [rows[p, base, pl.ds(r * 16, 16)] for r in range(NR)]
            for j in range(1, CTX):
                for r in range(NR):
                    vc[r] = vc[r] + rows[p, base + j, pl.ds(r * 16, 16)]
            inv = jnp.float32(1.0 / CTX)
            vc = [v * inv for v in vc]

            def dot_partial(row):
                t = [vc[r] * rows[p, row, pl.ds(r * 16, 16)]
                     for r in range(NR)]
                return (t[0] + t[1]) + (t[2] + t[3])

            slot = b * PSTRIDE
            pbuf[pl.ds(NEG * KSLOT + slot, 16)] = dot_partial(
                2 * IDS_PER_CHUNK + b)
            for k in range(NEG):
                pbuf[pl.ds(k * KSLOT + slot, 16)] = dot_partial(
                    IDS_PER_CHUNK + base + k)
            return carry_unused2

        lax.fori_loop(0, CB, b_body, 0, unroll=False)

        @pl.when(i >= 2)
        def _():
            pltpu.make_async_copy(pos_stage.at[p],
                                  pos_out.at[pl.ds(0, CB)], outsem.at[p]
                                  ).wait()
            pltpu.make_async_copy(neg_stage.at[p],
                                  neg_out.at[pl.ds(0, CB * NEG)],
                                  outsem.at[p]).wait()

        def treduce(k):
            vals = [plsc.load_gather(pbuf, [iota17 + (k * KSLOT + j)])
                    for j in range(16)]
            while len(vals) > 1:
                vals = [vals[2 * m] + vals[2 * m + 1]
                        for m in range(len(vals) // 2)]
            return vals[0]

        pos_stage[p] = treduce(NEG)
        for k in range(NEG):
            plsc.store_scatter(neg_stage.at[p], [iota20 + k], treduce(k))

        pltpu.async_copy(pos_stage.at[p], pos_out.at[pl.ds(b0, CB)],
                         outsem.at[p])
        pltpu.async_copy(neg_stage.at[p],
                         neg_out.at[pl.ds(b0 * NEG, CB * NEG)],
                         outsem.at[p])
        return carry_unused

    lax.fori_loop(0, NCHUNK, chunk_body, 0)

    for p in range(2):
        pltpu.make_async_copy(pos_stage.at[p], pos_out.at[pl.ds(0, CB)],
                              outsem.at[p]).wait()
        pltpu.make_async_copy(neg_stage.at[p],
                              neg_out.at[pl.ds(0, CB * NEG)],
                              outsem.at[p]).wait()


@jax.jit
def _sc_call(ctx_ids2, neg_ids2, tgt_ids2, in_embed, out_embed):
    mesh = plsc.VectorSubcoreMesh(core_axis_name="c", subcore_axis_name="s")
    f = functools.partial(
        pl.kernel,
        out_type=[
            jax.ShapeDtypeStruct((B,), jnp.float32),
            jax.ShapeDtypeStruct((B * NEG,), jnp.float32),
        ],
        mesh=mesh,
        compiler_params=pltpu.CompilerParams(
            needs_layout_passes=False, use_tc_tiling_on_sc=False),
        scratch_types=[
            pltpu.VMEM((BPW * CTX // QROWS, QROWS), jnp.int32),
            pltpu.VMEM((BPW * CTX // QROWS, QROWS), jnp.int32),
            pltpu.VMEM((NCHUNK, CB), jnp.int32),
            pltpu.VMEM((2, 2 * IDS_PER_CHUNK + CB, D), jnp.float32),
            pltpu.VMEM((NK * KSLOT,), jnp.float32),
            pltpu.VMEM((2, CB), jnp.float32),
            pltpu.VMEM((2, CB * NEG), jnp.float32),
            pltpu.SemaphoreType.DMA((2,)),
            pltpu.SemaphoreType.DMA((2,)),
        ],
    )(_sc_body)
    return f(ctx_ids2, neg_ids2, tgt_ids2, in_embed, out_embed)


def kernel(context_ids, target_ids, neg_ids, in_embed, out_embed):
    ctx2 = jnp.asarray(context_ids, jnp.int32).reshape(B * CTX // QROWS, QROWS)
    neg2 = jnp.asarray(neg_ids, jnp.int32).reshape(B * NEG // QROWS, QROWS)
    tgt2 = jnp.asarray(target_ids, jnp.int32).reshape(B // CB, CB)
    pos, neg_flat = _sc_call(ctx2, neg2, tgt2, in_embed, out_embed)
    return pos, neg_flat.reshape(B, NEG)
